# Initial kernel scaffold; baseline (speedup 1.0000x reference)
import functools

import jax
import jax.numpy as jnp
from jax import lax
from jax.experimental import pallas as pl
from jax.experimental.pallas import tpu as pltpu
from jax.experimental.pallas import tpu_sc as plsc

N_NODES = 10000
N_FEAT = 128
N_TRIPLES = 160000
N_EDGES = 2 * N_TRIPLES

NC = 2
NS = 16
LANES = 16

CHUNK = 128
EPT_HALF = N_EDGES // NC // NS
NCH_HALF = -(-EPT_HALF // CHUNK)
EPT_ALL = N_EDGES // NS
NCH_ALL = -(-EPT_ALL // CHUNK)

R_ACC = 10240
RPT = R_ACC // NS
DUMMY_ROW = N_NODES
SUB = 4
RSUB = RPT // SUB


def _sc_body(x_hbm, snd_hbm, rcv_hbm, rcvall_hbm, z2_hbm, z1_hbm,
             acc_out,
             idx_s, idx_r, idx_a, buf, ones_v, degv, stage,
             acc_sp, deg_sp, sem):
    c = lax.axis_index("c")
    s = lax.axis_index("s")

    pltpu.sync_copy(z2_hbm, acc_sp.at[pl.ds(s * RPT, RPT)])
    pltpu.sync_copy(z1_hbm, deg_sp.at[pl.ds(s * RPT, RPT)])

    pltpu.sync_copy(snd_hbm.at[c, s], idx_s)
    pltpu.sync_copy(rcv_hbm.at[c, s], idx_r)
    pltpu.sync_copy(rcvall_hbm.at[s], idx_a)
    for k in range(CHUNK // LANES):
        ones_v[pl.ds(k * LANES, LANES)] = jnp.full((LANES,), 1.0, jnp.float32)

    plsc.subcore_barrier()

    def row_body(j, carry):
        pltpu.async_copy(x_hbm.at[idx_s.at[j]], buf, sem).wait()
        pltpu.sync_copy(buf, acc_sp.at[idx_r.at[j]], add=True)
        return carry

    lax.fori_loop(0, NCH_HALF, row_body, 0)

    def deg_body(j, carry):
        pltpu.sync_copy(ones_v, deg_sp.at[idx_a.at[j]], add=True)
        return carry

    lax.fori_loop(0, NCH_ALL, deg_body, 0)

    plsc.subcore_barrier()

    pltpu.sync_copy(deg_sp.at[pl.ds(s * RPT, RPT)], degv)
    for k in range(RPT // LANES):
        d = degv[pl.ds(k * LANES, LANES)]
        degv[pl.ds(k * LANES, LANES)] = jnp.where(
            d > 0.0, 1.0 / jnp.maximum(d, 1.0), 0.0)

    for sub in range(SUB):
        row0 = s * RPT + sub * RSUB
        pltpu.sync_copy(acc_sp.at[pl.ds(row0, RSUB)], stage)

        def scale_body(r, carry):
            iv = degv[sub * RSUB + r]
            vec = jnp.full((LANES,), iv, jnp.float32)
            for k in range(N_FEAT // LANES):
                stage[r, pl.ds(k * LANES, LANES)] = (
                    stage[r, pl.ds(k * LANES, LANES)] * vec)
            return carry

        lax.fori_loop(0, RSUB, scale_body, 0)
        pltpu.sync_copy(stage, acc_out.at[c, s, pl.ds(sub * RSUB, RSUB)])


def _tc_merge_body(a_ref, o_ref):
    o_ref[...] = a_ref[0, 0] + a_ref[1, 0]


def kernel(x, triples):
    t = triples.T
    snd = jnp.concatenate([t[0], t[2]]).astype(jnp.int32)
    rcv = jnp.concatenate([t[2], t[0]]).astype(jnp.int32)

    pad_h = NCH_HALF * CHUNK - EPT_HALF
    pad_a = NCH_ALL * CHUNK - EPT_ALL

    snd_half = jnp.pad(
        snd.reshape(NC, NS, EPT_HALF), ((0, 0), (0, 0), (0, pad_h)),
        constant_values=0).reshape(NC, NS, NCH_HALF, CHUNK)
    rcv_half = jnp.pad(
        rcv.reshape(NC, NS, EPT_HALF), ((0, 0), (0, 0), (0, pad_h)),
        constant_values=DUMMY_ROW).reshape(NC, NS, NCH_HALF, CHUNK)
    rcv_all = jnp.pad(
        rcv.reshape(NS, EPT_ALL), ((0, 0), (0, pad_a)),
        constant_values=DUMMY_ROW).reshape(NS, NCH_ALL, CHUNK)

    z2 = jnp.zeros((RPT, N_FEAT), jnp.float32)
    z1 = jnp.zeros((RPT,), jnp.float32)

    mesh = plsc.VectorSubcoreMesh(core_axis_name="c", subcore_axis_name="s")
    acc = pl.kernel(
        _sc_body,
        out_type=jax.ShapeDtypeStruct((NC, NS, RPT, N_FEAT), jnp.float32),
        mesh=mesh,
        scratch_types=[
            pltpu.VMEM((NCH_HALF, CHUNK), jnp.int32),
            pltpu.VMEM((NCH_HALF, CHUNK), jnp.int32),
            pltpu.VMEM((NCH_ALL, CHUNK), jnp.int32),
            pltpu.VMEM((CHUNK, N_FEAT), jnp.float32),
            pltpu.VMEM((CHUNK,), jnp.float32),
            pltpu.VMEM((RPT,), jnp.float32),
            pltpu.VMEM((RSUB, N_FEAT), jnp.float32),
            pltpu.VMEM_SHARED((R_ACC, N_FEAT), jnp.float32),
            pltpu.VMEM_SHARED((R_ACC,), jnp.float32),
            pltpu.SemaphoreType.DMA,
        ],
    )(x, snd_half, rcv_half, rcv_all, z2, z1)

    out = pl.pallas_call(
        _tc_merge_body,
        grid=(NS,),
        in_specs=[pl.BlockSpec((NC, 1, RPT, N_FEAT), lambda i: (0, i, 0, 0))],
        out_specs=pl.BlockSpec((RPT, N_FEAT), lambda i: (i, 0)),
        out_shape=jax.ShapeDtypeStruct((N_NODES, N_FEAT), jnp.float32),
    )(acc)
    return out


# speedup vs baseline: 13.7028x; 13.7028x over previous
import jax
import jax.numpy as jnp
from jax import lax
from jax.experimental import pallas as pl
from jax.experimental.pallas import tpu as pltpu
from jax.experimental.pallas import tpu_sc as plsc

N_NODES = 10000
N_FEAT = 128
N_TRIPLES = 160000
N_EDGES = 2 * N_TRIPLES

NC = 2
NS = 16
LANES = 16
HFEAT = N_FEAT // NC

CHUNK = 128
EPT = N_EDGES // NS
NCH = -(-EPT // CHUNK)

R_ACC = 10240
RPT = R_ACC // NS
DUMMY_ROW = N_NODES
SUB = 4
RSUB = RPT // SUB


def _sc_body(x_hbm, snd_hbm, rcv_hbm, z2_hbm, z1_hbm,
             acc_out,
             idx_s, idx_r, buf, ones_v, degv, stage,
             acc_sp, deg_sp, sem):
    c = lax.axis_index("c")
    s = lax.axis_index("s")

    pltpu.sync_copy(z2_hbm, acc_sp.at[pl.ds(s * RPT, RPT)])
    pltpu.sync_copy(z1_hbm, deg_sp.at[pl.ds(s * RPT, RPT)])

    pltpu.sync_copy(snd_hbm.at[s], idx_s)
    pltpu.sync_copy(rcv_hbm.at[s], idx_r)
    for k in range(CHUNK // LANES):
        ones_v[pl.ds(k * LANES, LANES)] = jnp.full((LANES,), 1.0, jnp.float32)

    plsc.subcore_barrier()

    def edge_body(j, carry):
        pltpu.async_copy(x_hbm.at[c].at[idx_s.at[j]], buf, sem).wait()
        pltpu.sync_copy(buf, acc_sp.at[idx_r.at[j]], add=True)
        pltpu.sync_copy(ones_v, deg_sp.at[idx_r.at[j]], add=True)
        return carry

    lax.fori_loop(0, NCH, edge_body, 0)

    plsc.subcore_barrier()

    pltpu.sync_copy(deg_sp.at[pl.ds(s * RPT, RPT)], degv)
    for k in range(RPT // LANES):
        d = degv[pl.ds(k * LANES, LANES)]
        degv[pl.ds(k * LANES, LANES)] = jnp.where(
            d > 0.0, 1.0 / jnp.maximum(d, 1.0), 0.0)

    for sub in range(SUB):
        row0 = s * RPT + sub * RSUB
        pltpu.sync_copy(acc_sp.at[pl.ds(row0, RSUB)], stage)

        def scale_body(g, carry):
            iv16 = degv[pl.ds(sub * RSUB + g * LANES, LANES)]
            for l in range(LANES):
                r = g * LANES + l
                vec = jnp.full((LANES,), iv16[l], jnp.float32)
                for k in range(HFEAT // LANES):
                    stage[r, pl.ds(k * LANES, LANES)] = (
                        stage[r, pl.ds(k * LANES, LANES)] * vec)
            return carry

        lax.fori_loop(0, RSUB // LANES, scale_body, 0)
        pltpu.sync_copy(stage, acc_out.at[c, s, pl.ds(sub * RSUB, RSUB)])


def _tc_merge_body(a_ref, o_ref):
    o_ref[:, :HFEAT] = a_ref[0, 0]
    o_ref[:, HFEAT:] = a_ref[1, 0]


def kernel(x, triples):
    t = triples.T
    snd = jnp.concatenate([t[0], t[2]]).astype(jnp.int32)
    rcv = jnp.concatenate([t[2], t[0]]).astype(jnp.int32)

    pad = NCH * CHUNK - EPT

    snd_t = jnp.pad(
        snd.reshape(NS, EPT), ((0, 0), (0, pad)),
        constant_values=0).reshape(NS, NCH, CHUNK)
    rcv_t = jnp.pad(
        rcv.reshape(NS, EPT), ((0, 0), (0, pad)),
        constant_values=DUMMY_ROW).reshape(NS, NCH, CHUNK)

    x_cols = x.reshape(N_NODES, NC, HFEAT).transpose(1, 0, 2)

    z2 = jnp.zeros((RPT, HFEAT), jnp.float32)
    z1 = jnp.zeros((RPT,), jnp.float32)

    mesh = plsc.VectorSubcoreMesh(core_axis_name="c", subcore_axis_name="s")
    acc = pl.kernel(
        _sc_body,
        out_type=jax.ShapeDtypeStruct((NC, NS, RPT, HFEAT), jnp.float32),
        mesh=mesh,
        compiler_params=pltpu.CompilerParams(use_tc_tiling_on_sc=False),
        scratch_types=[
            pltpu.VMEM((NCH, CHUNK), jnp.int32),
            pltpu.VMEM((NCH, CHUNK), jnp.int32),
            pltpu.VMEM((CHUNK, HFEAT), jnp.float32),
            pltpu.VMEM((CHUNK,), jnp.float32),
            pltpu.VMEM((RPT,), jnp.float32),
            pltpu.VMEM((RSUB, HFEAT), jnp.float32),
            pltpu.VMEM_SHARED((R_ACC, HFEAT), jnp.float32),
            pltpu.VMEM_SHARED((R_ACC,), jnp.float32),
            pltpu.SemaphoreType.DMA,
        ],
    )(x_cols, snd_t, rcv_t, z2, z1)

    out = pl.pallas_call(
        _tc_merge_body,
        grid=(NS,),
        in_specs=[pl.BlockSpec((NC, 1, RPT, HFEAT), lambda i: (0, i, 0, 0))],
        out_specs=pl.BlockSpec((RPT, N_FEAT), lambda i: (i, 0)),
        out_shape=jax.ShapeDtypeStruct((N_NODES, N_FEAT), jnp.float32),
    )(acc)
    return out
